# baseline (speedup 1.0000x reference)
import numpy as np
import jax
import jax.numpy as jnp
from jax.experimental import pallas as pl

DIM = 2048
DEPTH = 7
PAR = 16
N_NODES = 2 ** (DEPTH + 1) - 1
N_INT = 2 ** DEPTH - 1
HALF = 128
NPAD = 256
UNROLL = 4
GW = UNROLL * HALF
NG = PAR // UNROLL
BT = 256

_DN_T = (((1,), (1,)), ((), ()))


def _tables():
    pos = np.array([n if n < N_INT else n + 1 for n in range(N_NODES)])
    anc = np.zeros((NPAD, HALF), np.float32)
    depth = np.full((NPAD,), -1.0, np.float32)
    for n in range(N_NODES):
        d = 0
        m = n
        while m != 0:
            parent = (m - 1) // 2
            anc[pos[n], parent] = 1.0 if (m - 1) % 2 else -1.0
            m = parent
            d += 1
        depth[pos[n]] = d
    anc_bd = np.zeros((2 * GW, GW), np.float32)
    dep_bd = np.zeros((1, 2 * GW), np.float32)
    for u in range(UNROLL):
        anc_bd[u * HALF:(u + 1) * HALF, u * HALF:(u + 1) * HALF] = \
            anc[:HALF]
        anc_bd[GW + u * HALF:GW + (u + 1) * HALF,
               u * HALF:(u + 1) * HALF] = anc[HALF:]
        dep_bd[0, u * HALF:(u + 1) * HALF] = depth[:HALF]
        dep_bd[0, GW + u * HALF:GW + (u + 1) * HALF] = depth[HALF:]
    return anc_bd, dep_bd


_ANC_BD, _DEP_BD = _tables()


def _split_halves(w3):
    w3 = jnp.concatenate([
        w3[:, :N_INT], jnp.zeros_like(w3[:, :1]), w3[:, N_INT:]], axis=1)
    w4 = w3.reshape(NG, UNROLL, NPAD, -1)
    return jnp.concatenate(
        [w4[:, :, :HALF].reshape(NG, GW, -1),
         w4[:, :, HALF:].reshape(NG, GW, -1)], axis=1)


def _fff_block(*refs):
    (xh_ref, xl_ref), refs = refs[:2], refs[2:]
    wi_refs, refs = refs[:NG], refs[NG:]
    wf_refs, refs = refs[:NG], refs[NG:]
    wl_refs, refs = refs[:NG], refs[NG:]
    wo_refs, refs = refs[:NG], refs[NG:]
    bi_ref, bl_ref, anc_ref, dep_ref, o_ref = refs
    xh = xh_ref[...]
    xl = xl_ref[...]
    anc = anc_ref[...]
    dep = dep_ref[...]
    acc = jnp.zeros((BT, DIM), jnp.float32)
    for q in range(NG):
        li = jax.lax.dot_general(xh, wi_refs[q][...], _DN_T,
                                 preferred_element_type=jnp.float32)
        li += jax.lax.dot_general(xh, wl_refs[q][...], _DN_T,
                                  preferred_element_type=jnp.float32)
        li += jax.lax.dot_general(xl, wi_refs[q][...], _DN_T,
                                  preferred_element_type=jnp.float32)
        li += bi_ref[q]
        ll = jax.lax.dot_general(xh, wf_refs[q][...], _DN_T,
                                 preferred_element_type=jnp.float32)
        ll += bl_ref[q]
        s = jnp.where(li > 0.0, 1.0, -1.0).astype(jnp.bfloat16)
        score = jax.lax.dot_general(s, anc, _DN_T,
                                    preferred_element_type=jnp.float32)
        act_i = li * jax.nn.sigmoid(li)
        act_l = ll * jax.nn.sigmoid(ll)
        act = jnp.concatenate([act_i, act_l], axis=1)
        act_m = jnp.where(score == dep, act, 0.0).astype(jnp.bfloat16)
        acc += jax.lax.dot_general(act_m, wo_refs[q][...],
                                   (((1,), (0,)), ((), ())),
                                   preferred_element_type=jnp.float32)
    o_ref[...] = acc


def kernel(oldx, W_in, b_in, W_out):
    x = oldx.reshape(-1, DIM)
    B = x.shape[0]
    x_hi = x.astype(jnp.bfloat16)
    x_lo = (x - x_hi.astype(jnp.float32)).astype(jnp.bfloat16)
    w_in_s = _split_halves(W_in.reshape(PAR, N_NODES, DIM))
    w_hi = w_in_s.astype(jnp.bfloat16)
    w_lo = (w_in_s[:, :GW] - w_hi[:, :GW].astype(jnp.float32)
            ).astype(jnp.bfloat16)
    w_int = [w_hi[q, :GW] for q in range(NG)]
    w_leaf = [w_hi[q, GW:] for q in range(NG)]
    w_low = [w_lo[q] for q in range(NG)]
    b_s = _split_halves(b_in.reshape(PAR, N_NODES, 1))[..., 0]
    b_int = b_s[:, None, :GW]
    b_leaf = b_s[:, None, GW:]
    anc = jnp.asarray(_ANC_BD).astype(jnp.bfloat16)
    dep = jnp.asarray(_DEP_BD)
    w_out_s = _split_halves(W_out.T.reshape(PAR, N_NODES, DIM)
                            ).astype(jnp.bfloat16)
    w_out = [w_out_s[q] for q in range(NG)]
    assert B % BT == 0
    full2 = pl.BlockSpec((GW, DIM), lambda i: (0, 0))
    full_out = pl.BlockSpec((2 * GW, DIM), lambda i: (0, 0))
    out = pl.pallas_call(
        _fff_block,
        grid=(B // BT,),
        in_specs=(
            [pl.BlockSpec((BT, DIM), lambda i: (i, 0))] * 2
            + [full2] * (3 * NG) + [full_out] * NG
            + [pl.BlockSpec((NG, 1, GW), lambda i: (0, 0, 0))] * 2
            + [pl.BlockSpec((2 * GW, GW), lambda i: (0, 0)),
               pl.BlockSpec((1, 2 * GW), lambda i: (0, 0))]
        ),
        out_specs=pl.BlockSpec((BT, DIM), lambda i: (i, 0)),
        out_shape=jax.ShapeDtypeStruct((B, DIM), jnp.float32),
    )(x_hi, x_lo, *w_int, *w_leaf, *w_low, *w_out, b_int, b_leaf, anc, dep)
    return out.reshape(oldx.shape)
